# baseline (speedup 1.0000x reference)
import functools

import jax
import jax.numpy as jnp
import numpy as np
from jax.experimental import pallas as pl

N_NODES = 50000
N_EDGES = 800000

M = 3
GAMMA = 1.0
LAMBDA3 = 1.3258
C = 4.8381
D_PARAM = 2.0417
COSTHETA0 = 0.0
N_PARAM = 22.956
BETA = 0.33675
LAMBDA2 = 1.3258
B_PARAM = 95.373
R_CUT = 3.0
D_CUT = 0.2
LAMBDA1 = 3.2394
A_PARAM = 3264.7

L3M = LAMBDA3 ** M
PI_2D = np.pi / (2 * D_CUT)


def _edge_stage1_kernel(rx, ry, rz, b_out, ib_out, fc_out, dfc_out):
    rr = rx[...] ** 2 + ry[...] ** 2 + rz[...] ** 2
    b = jnp.sqrt(rr)
    b_out[...] = b
    ib_out[...] = 1.0 / b
    sm = 0.5 - 0.5 * jnp.sin(PI_2D * (b - R_CUT))
    fc = jnp.where(b < R_CUT + D_CUT, sm, 0.0)
    fc_out[...] = jnp.where(b < R_CUT - D_CUT, 1.0, fc)
    d = -0.5 * PI_2D * jnp.cos(PI_2D * (b - R_CUT))
    mid = (b >= R_CUT - D_CUT) & (b < R_CUT + D_CUT)
    dfc_out[...] = jnp.where(mid, d, 0.0)


def _edge_stage3_kernel(zeta, b, ib, fc, dfc, w_out, gd_out, e_out):
    zp = zeta[...] + 1e-12
    t = N_PARAM * (np.log(BETA).astype(np.float32) + jnp.log(zp))
    sp = jnp.logaddexp(0.0, t)
    bb = jnp.exp(-sp / (2.0 * N_PARAM))
    sig = jax.nn.sigmoid(t)
    bv = b[...]
    f_rep = A_PARAM * jnp.exp(-LAMBDA1 * bv)
    f_att = -B_PARAM * jnp.exp(-LAMBDA2 * bv)
    fcv = fc[...]
    energy = fcv * (f_rep + bb * f_att) / 2.0
    e_out[...] = energy
    w_out[...] = fcv * f_att / 2.0 * (-bb * sig / (2.0 * zp))
    gd_out[...] = (dfc[...] * (f_rep + bb * f_att)
                   + fcv * (-LAMBDA1 * f_rep - LAMBDA2 * bb * f_att)) / 2.0


_ROWS = N_EDGES // 128


def _run_edge_stage1(r):
    rx = r[:, 0].reshape(_ROWS, 128)
    ry = r[:, 1].reshape(_ROWS, 128)
    rz = r[:, 2].reshape(_ROWS, 128)
    shp = jax.ShapeDtypeStruct((_ROWS, 128), jnp.float32)
    b, ib, fc, dfc = pl.pallas_call(
        _edge_stage1_kernel,
        out_shape=(shp, shp, shp, shp),
    )(rx, ry, rz)
    return (b.reshape(-1), ib.reshape(-1), fc.reshape(-1), dfc.reshape(-1))


def _run_edge_stage3(zeta, b, ib, fc, dfc):
    shp = jax.ShapeDtypeStruct((_ROWS, 128), jnp.float32)
    args = [x.reshape(_ROWS, 128) for x in (zeta, b, ib, fc, dfc)]
    w, gd, e = pl.pallas_call(
        _edge_stage3_kernel,
        out_shape=(shp, shp, shp),
    )(*args)
    return w.reshape(-1), gd.reshape(-1), e.reshape(-1)


def kernel(r, edge_index, lg_edge_index):
    lg_src, lg_dst = lg_edge_index[0], lg_edge_index[1]
    b, inv_b, fc, dfc = _run_edge_stage1(r)

    rs = r[lg_src]
    rd = r[lg_dst]
    bs = b[lg_src]
    bd = b[lg_dst]
    ibs = inv_b[lg_src]
    ibd = inv_b[lg_dst]
    fcs = fc[lg_src]
    dot = jnp.sum(rs * rd, axis=1)
    cos_raw = -dot * ibs * ibd
    cos = jnp.clip(cos_raw, -1.0, 1.0)
    db = bd - bs
    h = L3M * db ** 3
    eh = jnp.exp(h)
    q = D_PARAM ** 2 + (cos - COSTHETA0) ** 2
    g = GAMMA * (1.0 + C ** 2 / D_PARAM ** 2 - C ** 2 / q)
    zeta_k = fcs * g * eh
    zeta = jax.ops.segment_sum(zeta_k, lg_dst, num_segments=N_EDGES)

    w, gd_scalar, energy = _run_edge_stage3(zeta, b, inv_b, fc, dfc)
    E = jnp.sum(energy)
    gdir = (gd_scalar * inv_b)[:, None] * r

    wd = w[lg_dst]
    dfcs = dfc[lg_src]
    gp = GAMMA * (2.0 * C ** 2) * (cos - COSTHETA0) / q ** 2
    inrange = (cos_raw >= -1.0) & (cos_raw <= 1.0)
    A_c = jnp.where(inrange, fcs * gp * eh, 0.0)
    P = zeta_k * (3.0 * L3M * db ** 2)
    dz_dbs = dfcs * g * eh - P
    ibsd = ibs * ibd
    c_d = wd[:, None] * (A_c[:, None] * (-rs * ibsd[:, None] - (cos * ibd ** 2)[:, None] * rd)
                         + (P * ibd)[:, None] * rd)
    c_s = wd[:, None] * (A_c[:, None] * (-rd * ibsd[:, None] - (cos * ibs ** 2)[:, None] * rs)
                         + (dz_dbs * ibs)[:, None] * rs)
    gscat = (jax.ops.segment_sum(c_d, lg_dst, num_segments=N_EDGES)
             + jax.ops.segment_sum(c_s, lg_src, num_segments=N_EDGES))
    dE_dr = gdir + gscat

    src, dst = edge_index[0], edge_index[1]
    forces = (jax.ops.segment_sum(dE_dr, src, num_segments=N_NODES)
              - jax.ops.segment_sum(dE_dr, dst, num_segments=N_NODES))
    return (E, forces)


# speedup vs baseline: 76.6883x; 76.6883x over previous
import functools

import jax
import jax.numpy as jnp
import numpy as np
from jax import lax
from jax.experimental import pallas as pl
from jax.experimental.pallas import tpu as pltpu
from jax.experimental.pallas import tpu_sc as plsc

N_NODES = 50000
N_EDGES = 800000
N_LG = 12000000

M = 3
GAMMA = 1.0
LAMBDA3 = 1.3258
C = 4.8381
D_PARAM = 2.0417
COSTHETA0 = 0.0
N_PARAM = 22.956
BETA = 0.33675
LAMBDA2 = 1.3258
B_PARAM = 95.373
R_CUT = 3.0
D_CUT = 0.2
LAMBDA1 = 3.2394
A_PARAM = 3264.7

L3M = LAMBDA3 ** M
PI_2D = np.pi / (2 * D_CUT)
K1 = GAMMA * (1.0 + C ** 2 / D_PARAM ** 2)
C2G = GAMMA * C ** 2
D2 = D_PARAM ** 2

_ROWS = N_EDGES // 128
CHUNK = 128
NLG_CHUNKS = N_LG // CHUNK
NW = 32
ITER_FWD = -(-NLG_CHUNKS // NW)
NDIR_CHUNKS = N_EDGES // CHUNK
ITER_DIR = -(-NDIR_CHUNKS // NW)
EPC = N_EDGES // 16
NPAD = 50048
NPC = NPAD // 16

TW = 16


def _tc1_body(rx, ry, rz, b_o, ib_o, fc_o, dfc_o):
    rr = rx[...] ** 2 + ry[...] ** 2 + rz[...] ** 2
    b = jnp.sqrt(rr)
    b_o[...] = b
    ib_o[...] = 1.0 / b
    sm = 0.5 - 0.5 * jnp.sin(PI_2D * (b - R_CUT))
    fc = jnp.where(b < R_CUT + D_CUT, sm, 0.0)
    fc_o[...] = jnp.where(b < R_CUT - D_CUT, 1.0, fc)
    d = -0.5 * PI_2D * jnp.cos(PI_2D * (b - R_CUT))
    mid = (b >= R_CUT - D_CUT) & (b < R_CUT + D_CUT)
    dfc_o[...] = jnp.where(mid, d, 0.0)


def _tc2_body(z0, z1, b, ib, fc, dfc, w_o, w2_o, e_o):
    zeta = z0[...] + z1[...]
    zp = zeta + 1e-12
    t = N_PARAM * (np.float32(np.log(BETA)) + jnp.log(zp))
    sp = jnp.logaddexp(0.0, t)
    bb = jnp.exp(-sp / (2.0 * N_PARAM))
    sig = jax.nn.sigmoid(t)
    bv = b[...]
    f_rep = A_PARAM * jnp.exp(-LAMBDA1 * bv)
    f_att = -B_PARAM * jnp.exp(-LAMBDA2 * bv)
    fcv = fc[...]
    energy = fcv * (f_rep + bb * f_att) / 2.0
    e_o[...] = jnp.sum(energy)[None, None]
    w_o[...] = fcv * f_att / 2.0 * (-bb * sig / (2.0 * zp))
    gd = (dfc[...] * (f_rep + bb * f_att)
          + fcv * (-LAMBDA1 * f_rep - LAMBDA2 * bb * f_att)) / 2.0
    w2_o[...] = gd * ib[...]


def _tc3_body(f2, out):
    out[...] = f2[0] + f2[1]


_shp = jax.ShapeDtypeStruct((_ROWS, 128), jnp.float32)

_tc1 = pl.pallas_call(_tc1_body, out_shape=(_shp, _shp, _shp, _shp))
_tc2 = pl.pallas_call(
    _tc2_body,
    out_shape=(_shp, _shp, jax.ShapeDtypeStruct((1, 1), jnp.float32)),
)
_tc3 = pl.pallas_call(
    _tc3_body,
    out_shape=jax.ShapeDtypeStruct((NPAD * 4 // 128, 128), jnp.float32),
)


def _sc_fwd_body(table, lg_src, lg_dst, zeros_z, zout,
                 idx_s, idx_d, recs_s, recs_d, zvals, zacc, sem):
    cid = lax.axis_index("c")
    sid = lax.axis_index("s")
    wid = sid * 2 + cid
    pltpu.sync_copy(zeros_z.at[pl.ds(sid * EPC, EPC)],
                    zacc.at[pl.ds(sid * EPC, EPC)])
    plsc.subcore_barrier()
    lane = lax.iota(jnp.int32, 16)

    def chunk_body(i, carry):
        cnum = wid + i * NW

        @pl.when(cnum < NLG_CHUNKS)
        def _():
            base = cnum * CHUNK
            cp1 = pltpu.async_copy(lg_src.at[pl.ds(base, CHUNK)], idx_s, sem)
            cp2 = pltpu.async_copy(lg_dst.at[pl.ds(base, CHUNK)], idx_d, sem)
            cp1.wait()
            cp2.wait()
            g1 = pltpu.async_copy(table.at[idx_s], recs_s, sem)
            g2 = pltpu.async_copy(table.at[idx_d], recs_d, sem)
            g1.wait()
            g2.wait()
            for grp in range(CHUNK // 16):
                rows = lane + grp * 16

                def lds(ref, c_):
                    return plsc.load_gather(
                        ref, [rows, jnp.full((16,), c_, jnp.int32)])

                rsx, rsy, rsz = lds(recs_s, 0), lds(recs_s, 1), lds(recs_s, 2)
                bs, ibs, fcs = lds(recs_s, 3), lds(recs_s, 4), lds(recs_s, 5)
                rdx, rdy, rdz = lds(recs_d, 0), lds(recs_d, 1), lds(recs_d, 2)
                bd, ibd = lds(recs_d, 3), lds(recs_d, 4)
                dot = rsx * rdx + rsy * rdy + rsz * rdz
                cosr = -dot * ibs * ibd
                cos = jnp.clip(cosr, -1.0, 1.0)
                dbv = bd - bs
                eh = jnp.exp(L3M * dbv * dbv * dbv)
                cc = cos - COSTHETA0
                q = D2 + cc * cc
                gang = K1 - C2G / q
                zvals[pl.ds(grp * 16, 16)] = fcs * gang * eh
            pltpu.sync_copy(zvals, zacc.at[idx_d], add=True)
        return carry

    lax.fori_loop(0, ITER_FWD, chunk_body, 0)
    plsc.subcore_barrier()
    pltpu.sync_copy(zacc.at[pl.ds(sid * EPC, EPC)],
                    zout.at[pl.ds(cid * N_EDGES + sid * EPC, EPC)])


_sc_fwd = functools.partial(
    pl.kernel,
    out_type=jax.ShapeDtypeStruct((2 * N_EDGES,), jnp.float32),
    mesh=plsc.VectorSubcoreMesh(core_axis_name="c", subcore_axis_name="s"),
    scratch_types=[
        pltpu.VMEM((CHUNK,), jnp.int32),
        pltpu.VMEM((CHUNK,), jnp.int32),
        pltpu.VMEM((CHUNK, TW), jnp.float32),
        pltpu.VMEM((CHUNK, TW), jnp.float32),
        pltpu.VMEM((CHUNK,), jnp.float32),
        pltpu.VMEM_SHARED((N_EDGES,), jnp.float32),
        pltpu.SemaphoreType.DMA,
    ],
    compiler_params=pltpu.CompilerParams(
        needs_layout_passes=False, use_tc_tiling_on_sc=False),
)(_sc_fwd_body)


def _sc_bwd_body(table, lg_src, lg_dst, zeros_f, fout,
                 idx_s, idx_d, recs_s, recs_d,
                 vals, idxs, facc, sem):
    cid = lax.axis_index("c")
    sid = lax.axis_index("s")
    wid = sid * 2 + cid
    pltpu.sync_copy(zeros_f.at[pl.ds(sid * NPC * 4, NPC * 4)],
                    facc.at[pl.ds(sid * NPC * 4, NPC * 4)])
    plsc.subcore_barrier()
    lane = lax.iota(jnp.int32, 16)

    def lg_body(i, carry):
        cnum = wid + i * NW

        @pl.when(cnum < NLG_CHUNKS)
        def _():
            base = cnum * CHUNK
            cp1 = pltpu.async_copy(lg_src.at[pl.ds(base, CHUNK)], idx_s, sem)
            cp2 = pltpu.async_copy(lg_dst.at[pl.ds(base, CHUNK)], idx_d, sem)
            cp1.wait()
            cp2.wait()
            g1 = pltpu.async_copy(table.at[idx_s], recs_s, sem)
            g2 = pltpu.async_copy(table.at[idx_d], recs_d, sem)
            g1.wait()
            g2.wait()
            for grp in range(CHUNK // 16):
                rows = lane + grp * 16
                sl = pl.ds(grp * 16, 16)

                def lds(ref, c_):
                    return plsc.load_gather(
                        ref, [rows, jnp.full((16,), c_, jnp.int32)])

                rsx, rsy, rsz = lds(recs_s, 0), lds(recs_s, 1), lds(recs_s, 2)
                bs, ibs, fcs = lds(recs_s, 3), lds(recs_s, 4), lds(recs_s, 5)
                dfcs = lds(recs_s, 6)
                rdx, rdy, rdz = lds(recs_d, 0), lds(recs_d, 1), lds(recs_d, 2)
                bd, ibd, wd = lds(recs_d, 3), lds(recs_d, 4), lds(recs_d, 7)
                ns_d = plsc.bitcast(lds(recs_d, 8), jnp.int32) * 4
                nd_d = plsc.bitcast(lds(recs_d, 9), jnp.int32) * 4
                ns_s = plsc.bitcast(lds(recs_s, 8), jnp.int32) * 4
                nd_s = plsc.bitcast(lds(recs_s, 9), jnp.int32) * 4
                dot = rsx * rdx + rsy * rdy + rsz * rdz
                cosr = -dot * ibs * ibd
                cos = jnp.clip(cosr, -1.0, 1.0)
                dbv = bd - bs
                eh = jnp.exp(L3M * dbv * dbv * dbv)
                cc = cos - COSTHETA0
                q = D2 + cc * cc
                gang = K1 - C2G / q
                gp = 2.0 * C2G * cc / (q * q)
                inr = (cosr >= -1.0) & (cosr <= 1.0)
                A_c = jnp.where(inr, fcs * gp * eh, 0.0)
                zk = fcs * gang * eh
                P = zk * (3.0 * L3M) * dbv * dbv
                dzbs = dfcs * gang * eh - P
                alpha = -wd * A_c * ibs * ibd
                beta_d = wd * (P * ibd - A_c * cos * ibd * ibd)
                beta_s = wd * (dzbs * ibs - A_c * cos * ibs * ibs)
                cdx = alpha * rsx + beta_d * rdx
                cdy = alpha * rsy + beta_d * rdy
                cdz = alpha * rsz + beta_d * rdz
                csx = alpha * rdx + beta_s * rsx
                csy = alpha * rdy + beta_s * rsy
                csz = alpha * rdz + beta_s * rsz
                for j, (vv, ii) in enumerate((
                        (cdx, ns_d), (cdy, ns_d), (cdz, ns_d),
                        (-cdx, nd_d), (-cdy, nd_d), (-cdz, nd_d),
                        (csx, ns_s), (csy, ns_s), (csz, ns_s),
                        (-csx, nd_s), (-csy, nd_s), (-csz, nd_s))):
                    vals[j, sl] = vv
                    idxs[j, sl] = ii + (j % 3)
            for j in range(12):
                pltpu.sync_copy(vals.at[j], facc.at[idxs.at[j]], add=True)
        return carry

    lax.fori_loop(0, ITER_FWD, lg_body, 0)

    def dir_body(i, carry):
        cnum = wid + i * NW

        @pl.when(cnum < NDIR_CHUNKS)
        def _():
            base = cnum * CHUNK
            pltpu.async_copy(table.at[pl.ds(base, CHUNK)], recs_s, sem).wait()
            for grp in range(CHUNK // 16):
                rows = lane + grp * 16
                sl = pl.ds(grp * 16, 16)

                def lds(c_):
                    return plsc.load_gather(
                        recs_s, [rows, jnp.full((16,), c_, jnp.int32)])

                rx, ry, rz, w2 = lds(0), lds(1), lds(2), lds(10)
                ns = plsc.bitcast(lds(8), jnp.int32) * 4
                nd = plsc.bitcast(lds(9), jnp.int32) * 4
                cx, cy, cz = w2 * rx, w2 * ry, w2 * rz
                for j, (vv, ii) in enumerate((
                        (cx, ns), (cy, ns), (cz, ns),
                        (-cx, nd), (-cy, nd), (-cz, nd))):
                    vals[j, sl] = vv
                    idxs[j, sl] = ii + (j % 3)
            for j in range(6):
                pltpu.sync_copy(vals.at[j], facc.at[idxs.at[j]], add=True)
        return carry

    lax.fori_loop(0, ITER_DIR, dir_body, 0)
    plsc.subcore_barrier()
    pltpu.sync_copy(facc.at[pl.ds(sid * NPC * 4, NPC * 4)],
                    fout.at[pl.ds(cid * NPAD * 4 + sid * NPC * 4, NPC * 4)])


_sc_bwd = functools.partial(
    pl.kernel,
    out_type=jax.ShapeDtypeStruct((2 * NPAD * 4,), jnp.float32),
    mesh=plsc.VectorSubcoreMesh(core_axis_name="c", subcore_axis_name="s"),
    scratch_types=[
        pltpu.VMEM((CHUNK,), jnp.int32),
        pltpu.VMEM((CHUNK,), jnp.int32),
        pltpu.VMEM((CHUNK, TW), jnp.float32),
        pltpu.VMEM((CHUNK, TW), jnp.float32),
        pltpu.VMEM((12, CHUNK), jnp.float32),
        pltpu.VMEM((12, CHUNK), jnp.int32),
        pltpu.VMEM_SHARED((NPAD * 4,), jnp.float32),
        pltpu.SemaphoreType.DMA,
    ],
    compiler_params=pltpu.CompilerParams(
        needs_layout_passes=False, use_tc_tiling_on_sc=False),
)(_sc_bwd_body)


def kernel(r, edge_index, lg_edge_index):
    lg_src = lg_edge_index[0]
    lg_dst = lg_edge_index[1]
    rxf, ryf, rzf = r[:, 0], r[:, 1], r[:, 2]
    rx2 = rxf.reshape(_ROWS, 128)
    ry2 = ryf.reshape(_ROWS, 128)
    rz2 = rzf.reshape(_ROWS, 128)
    b2, ib2, fc2, dfc2 = _tc1(rx2, ry2, rz2)
    flat = lambda a: a.reshape(-1)
    nsf = lax.bitcast_convert_type(edge_index[0], jnp.float32)
    ndf = lax.bitcast_convert_type(edge_index[1], jnp.float32)
    zcol = jnp.zeros((N_EDGES,), jnp.float32)
    bf, ibf, fcf, dfcf = flat(b2), flat(ib2), flat(fc2), flat(dfc2)
    table_f = jnp.stack(
        [rxf, ryf, rzf, bf, ibf, fcf, dfcf, zcol, nsf, ndf,
         zcol, zcol, zcol, zcol, zcol, zcol], axis=1)
    zeros_z = jnp.zeros((N_EDGES,), jnp.float32)
    zpart = _sc_fwd(table_f, lg_src, lg_dst, zeros_z).reshape(2, N_EDGES)
    z0 = zpart[0].reshape(_ROWS, 128)
    z1 = zpart[1].reshape(_ROWS, 128)
    w2d, w22d, e11 = _tc2(z0, z1, b2, ib2, fc2, dfc2)
    table_b = jnp.stack(
        [rxf, ryf, rzf, bf, ibf, fcf, dfcf, flat(w2d), nsf, ndf,
         flat(w22d), zcol, zcol, zcol, zcol, zcol], axis=1)
    zeros_f = jnp.zeros((NPAD * 4,), jnp.float32)
    fpart = _sc_bwd(table_b, lg_src, lg_dst, zeros_f)
    fsum = _tc3(fpart.reshape(2, NPAD * 4 // 128, 128))
    forces = fsum.reshape(NPAD, 4)[:N_NODES, :3]
    return (e11[0, 0], forces)


# speedup vs baseline: 174.8201x; 2.2796x over previous
import functools

import jax
import jax.numpy as jnp
import numpy as np
from jax import lax
from jax.experimental import pallas as pl
from jax.experimental.pallas import tpu as pltpu
from jax.experimental.pallas import tpu_sc as plsc

N_NODES = 50000
N_EDGES = 800000
N_LG = 12000000

M = 3
GAMMA = 1.0
LAMBDA3 = 1.3258
C = 4.8381
D_PARAM = 2.0417
COSTHETA0 = 0.0
N_PARAM = 22.956
BETA = 0.33675
LAMBDA2 = 1.3258
B_PARAM = 95.373
R_CUT = 3.0
D_CUT = 0.2
LAMBDA1 = 3.2394
A_PARAM = 3264.7

L3M = LAMBDA3 ** M
PI_2D = np.pi / (2 * D_CUT)
K1 = GAMMA * (1.0 + C ** 2 / D_PARAM ** 2)
C2G = GAMMA * C ** 2
D2 = D_PARAM ** 2

_ROWS = N_EDGES // 128
CHUNK = 128
NLG_CHUNKS = N_LG // CHUNK
NW = 32
ITER_FWD = -(-NLG_CHUNKS // NW)
NDIR_CHUNKS = N_EDGES // CHUNK
ITER_DIR = -(-NDIR_CHUNKS // NW)
EPC = N_EDGES // 16
NPAD = 50048
NPC = NPAD // 16

TW = 16


def _tc1_body(rx, ry, rz, b_o, ib_o, fc_o, dfc_o):
    rr = rx[...] ** 2 + ry[...] ** 2 + rz[...] ** 2
    b = jnp.sqrt(rr)
    b_o[...] = b
    ib_o[...] = 1.0 / b
    sm = 0.5 - 0.5 * jnp.sin(PI_2D * (b - R_CUT))
    fc = jnp.where(b < R_CUT + D_CUT, sm, 0.0)
    fc_o[...] = jnp.where(b < R_CUT - D_CUT, 1.0, fc)
    d = -0.5 * PI_2D * jnp.cos(PI_2D * (b - R_CUT))
    mid = (b >= R_CUT - D_CUT) & (b < R_CUT + D_CUT)
    dfc_o[...] = jnp.where(mid, d, 0.0)


def _tc2_body(z0, z1, b, ib, fc, dfc, w_o, w2_o, e_o):
    zeta = z0[...] + z1[...]
    zp = zeta + 1e-12
    t = N_PARAM * (np.float32(np.log(BETA)) + jnp.log(zp))
    sp = jnp.logaddexp(0.0, t)
    bb = jnp.exp(-sp / (2.0 * N_PARAM))
    sig = jax.nn.sigmoid(t)
    bv = b[...]
    f_rep = A_PARAM * jnp.exp(-LAMBDA1 * bv)
    f_att = -B_PARAM * jnp.exp(-LAMBDA2 * bv)
    fcv = fc[...]
    energy = fcv * (f_rep + bb * f_att) / 2.0
    e_o[...] = jnp.sum(energy)[None, None]
    w_o[...] = fcv * f_att / 2.0 * (-bb * sig / (2.0 * zp))
    gd = (dfc[...] * (f_rep + bb * f_att)
          + fcv * (-LAMBDA1 * f_rep - LAMBDA2 * bb * f_att)) / 2.0
    w2_o[...] = gd * ib[...]


def _tc3_body(f2, out):
    out[...] = f2[0] + f2[1]


_shp = jax.ShapeDtypeStruct((_ROWS, 128), jnp.float32)

_tc1 = pl.pallas_call(_tc1_body, out_shape=(_shp, _shp, _shp, _shp))
_tc2 = pl.pallas_call(
    _tc2_body,
    out_shape=(_shp, _shp, jax.ShapeDtypeStruct((1, 1), jnp.float32)),
)
_tc3 = pl.pallas_call(
    _tc3_body,
    out_shape=jax.ShapeDtypeStruct((NPAD * 4 // 128, 128), jnp.float32),
)


def _sc_fwd_body(table, lg_src, lg_dst, zeros_z, zout, *refs):
    (idx_s0, idx_d0, recs_s0, recs_d0, zvals0, sidx0,
     idx_s1, idx_d1, recs_s1, recs_d1, zvals1, sidx1,
     zacc, semi0, semi1, semg0, semg1, semsc0, semsc1) = refs
    idx_s = (idx_s0, idx_s1)
    idx_d = (idx_d0, idx_d1)
    recs_s = (recs_s0, recs_s1)
    recs_d = (recs_d0, recs_d1)
    zvals = (zvals0, zvals1)
    sidx = (sidx0, sidx1)
    semi = (semi0, semi1)
    semg = (semg0, semg1)
    semsc = (semsc0, semsc1)
    cid = lax.axis_index("c")
    sid = lax.axis_index("s")
    wid = sid * 2 + cid
    pltpu.sync_copy(zeros_z.at[pl.ds(sid * EPC, EPC)],
                    zacc.at[pl.ds(sid * EPC, EPC)])
    plsc.subcore_barrier()
    lane = lax.iota(jnp.int32, 16)

    def cnum_of(j):
        return wid + j * NW

    def idx_issue(j, b):
        @pl.when((j >= 0) & (cnum_of(j) < NLG_CHUNKS))
        def _():
            base = cnum_of(j) * CHUNK
            pltpu.async_copy(lg_src.at[pl.ds(base, CHUNK)], idx_s[b], semi[b])
            pltpu.async_copy(lg_dst.at[pl.ds(base, CHUNK)], idx_d[b], semi[b])

    def idx_wait(j, b):
        @pl.when((j >= 0) & (cnum_of(j) < NLG_CHUNKS))
        def _():
            pltpu.make_async_copy(lg_src.at[pl.ds(0, CHUNK)], idx_s[b], semi[b]).wait()
            pltpu.make_async_copy(lg_dst.at[pl.ds(0, CHUNK)], idx_d[b], semi[b]).wait()

    def gather_issue(j, b):
        @pl.when((j >= 0) & (cnum_of(j) < NLG_CHUNKS))
        def _():
            pltpu.async_copy(table.at[idx_s[b]], recs_s[b], semg[b])
            pltpu.async_copy(table.at[idx_d[b]], recs_d[b], semg[b])

    def gather_wait(j, b):
        @pl.when((j >= 0) & (cnum_of(j) < NLG_CHUNKS))
        def _():
            pltpu.make_async_copy(table.at[idx_s[b]], recs_s[b], semg[b]).wait()
            pltpu.make_async_copy(table.at[idx_d[b]], recs_d[b], semg[b]).wait()

    def scatter_drain(j, b):
        @pl.when((j >= 0) & (cnum_of(j) < NLG_CHUNKS))
        def _():
            pltpu.make_async_copy(zvals[b], zacc.at[sidx[b]], semsc[b]).wait()

    def compute_scatter(j, b):
        @pl.when((j >= 0) & (cnum_of(j) < NLG_CHUNKS))
        def _():
            for grp in range(CHUNK // 16):
                rows = lane + grp * 16
                sl = pl.ds(grp * 16, 16)

                def lds(ref, c_):
                    return plsc.load_gather(
                        ref, [rows, jnp.full((16,), c_, jnp.int32)])

                rsx, rsy, rsz = (lds(recs_s[b], 0), lds(recs_s[b], 1),
                                 lds(recs_s[b], 2))
                bs, ibs, fcs = (lds(recs_s[b], 3), lds(recs_s[b], 4),
                                lds(recs_s[b], 5))
                rdx, rdy, rdz = (lds(recs_d[b], 0), lds(recs_d[b], 1),
                                 lds(recs_d[b], 2))
                bd, ibd = lds(recs_d[b], 3), lds(recs_d[b], 4)
                sidx[b][sl] = idx_d[b][sl]
                dot = rsx * rdx + rsy * rdy + rsz * rdz
                cosr = -dot * ibs * ibd
                cos = jnp.clip(cosr, -1.0, 1.0)
                dbv = bd - bs
                eh = jnp.exp(L3M * dbv * dbv * dbv)
                cc = cos - COSTHETA0
                q = D2 + cc * cc
                gang = K1 - C2G / q
                zvals[b][sl] = fcs * gang * eh
            pltpu.async_copy(zvals[b], zacc.at[sidx[b]], semsc[b], add=True)

    idx_issue(0, 0)
    idx_issue(1, 1)
    idx_wait(0, 0)
    gather_issue(0, 0)

    def loop_body(jj, carry):
        for b in (0, 1):
            j = 2 * jj + b
            o = 1 - b
            scatter_drain(j - 2, b)
            gather_wait(j, b)
            idx_wait(j + 1, o)
            gather_issue(j + 1, o)
            compute_scatter(j, b)
            idx_issue(j + 2, b)
        return carry

    lax.fori_loop(0, ITER_FWD // 2, loop_body, 0)
    scatter_drain(ITER_FWD - 2, 0)
    scatter_drain(ITER_FWD - 1, 1)
    plsc.subcore_barrier()
    pltpu.sync_copy(zacc.at[pl.ds(sid * EPC, EPC)],
                    zout.at[pl.ds(cid * N_EDGES + sid * EPC, EPC)])


_sc_fwd = functools.partial(
    pl.kernel,
    out_type=jax.ShapeDtypeStruct((2 * N_EDGES,), jnp.float32),
    mesh=plsc.VectorSubcoreMesh(core_axis_name="c", subcore_axis_name="s"),
    scratch_types=[
        pltpu.VMEM((CHUNK,), jnp.int32),
        pltpu.VMEM((CHUNK,), jnp.int32),
        pltpu.VMEM((CHUNK, TW), jnp.float32),
        pltpu.VMEM((CHUNK, TW), jnp.float32),
        pltpu.VMEM((CHUNK,), jnp.float32),
        pltpu.VMEM((CHUNK,), jnp.int32),
        pltpu.VMEM((CHUNK,), jnp.int32),
        pltpu.VMEM((CHUNK,), jnp.int32),
        pltpu.VMEM((CHUNK, TW), jnp.float32),
        pltpu.VMEM((CHUNK, TW), jnp.float32),
        pltpu.VMEM((CHUNK,), jnp.float32),
        pltpu.VMEM((CHUNK,), jnp.int32),
        pltpu.VMEM_SHARED((N_EDGES,), jnp.float32),
        pltpu.SemaphoreType.DMA,
        pltpu.SemaphoreType.DMA,
        pltpu.SemaphoreType.DMA,
        pltpu.SemaphoreType.DMA,
        pltpu.SemaphoreType.DMA,
        pltpu.SemaphoreType.DMA,
    ],
    compiler_params=pltpu.CompilerParams(
        needs_layout_passes=False, use_tc_tiling_on_sc=False),
)(_sc_fwd_body)


def _sc_bwd_body(table, lg_src, lg_dst, zeros_f, fout, *refs):
    (idx_s0, idx_d0, recs_s0, recs_d0, vals0, idxs0,
     idx_s1, idx_d1, recs_s1, recs_d1, vals1, idxs1,
     facc, semi0, semi1, semg0, semg1, semsc0, semsc1) = refs
    idx_s = (idx_s0, idx_s1)
    idx_d = (idx_d0, idx_d1)
    recs_s = (recs_s0, recs_s1)
    recs_d = (recs_d0, recs_d1)
    vals = (vals0, vals1)
    idxs = (idxs0, idxs1)
    semi = (semi0, semi1)
    semg = (semg0, semg1)
    semsc = (semsc0, semsc1)
    cid = lax.axis_index("c")
    sid = lax.axis_index("s")
    wid = sid * 2 + cid
    pltpu.sync_copy(zeros_f.at[pl.ds(sid * NPC * 4, NPC * 4)],
                    facc.at[pl.ds(sid * NPC * 4, NPC * 4)])
    plsc.subcore_barrier()
    lane = lax.iota(jnp.int32, 16)

    def cnum_of(j):
        return wid + j * NW

    def idx_issue(j, b):
        @pl.when((j >= 0) & (cnum_of(j) < NLG_CHUNKS))
        def _():
            base = cnum_of(j) * CHUNK
            pltpu.async_copy(lg_src.at[pl.ds(base, CHUNK)], idx_s[b], semi[b])
            pltpu.async_copy(lg_dst.at[pl.ds(base, CHUNK)], idx_d[b], semi[b])

    def idx_wait(j, b):
        @pl.when((j >= 0) & (cnum_of(j) < NLG_CHUNKS))
        def _():
            pltpu.make_async_copy(lg_src.at[pl.ds(0, CHUNK)], idx_s[b], semi[b]).wait()
            pltpu.make_async_copy(lg_dst.at[pl.ds(0, CHUNK)], idx_d[b], semi[b]).wait()

    def gather_issue(j, b):
        @pl.when((j >= 0) & (cnum_of(j) < NLG_CHUNKS))
        def _():
            pltpu.async_copy(table.at[idx_s[b]], recs_s[b], semg[b])
            pltpu.async_copy(table.at[idx_d[b]], recs_d[b], semg[b])

    def gather_wait(j, b):
        @pl.when((j >= 0) & (cnum_of(j) < NLG_CHUNKS))
        def _():
            pltpu.make_async_copy(table.at[idx_s[b]], recs_s[b], semg[b]).wait()
            pltpu.make_async_copy(table.at[idx_d[b]], recs_d[b], semg[b]).wait()

    def scatter_drain(j, b, nstreams=12):
        @pl.when((j >= 0) & (cnum_of(j) < NLG_CHUNKS))
        def _():
            for k in range(nstreams):
                pltpu.make_async_copy(
                    vals[b].at[k], facc.at[idxs[b].at[k]], semsc[b]).wait()

    def compute_scatter(j, b):
        @pl.when((j >= 0) & (cnum_of(j) < NLG_CHUNKS))
        def _():
            for grp in range(CHUNK // 16):
                rows = lane + grp * 16
                sl = pl.ds(grp * 16, 16)

                def lds(ref, c_):
                    return plsc.load_gather(
                        ref, [rows, jnp.full((16,), c_, jnp.int32)])

                rsx, rsy, rsz = (lds(recs_s[b], 0), lds(recs_s[b], 1),
                                 lds(recs_s[b], 2))
                bs, ibs, fcs = (lds(recs_s[b], 3), lds(recs_s[b], 4),
                                lds(recs_s[b], 5))
                dfcs = lds(recs_s[b], 6)
                rdx, rdy, rdz = (lds(recs_d[b], 0), lds(recs_d[b], 1),
                                 lds(recs_d[b], 2))
                bd, ibd, wd = (lds(recs_d[b], 3), lds(recs_d[b], 4),
                               lds(recs_d[b], 7))
                ns_d = plsc.bitcast(lds(recs_d[b], 8), jnp.int32) * 4
                nd_d = plsc.bitcast(lds(recs_d[b], 9), jnp.int32) * 4
                ns_s = plsc.bitcast(lds(recs_s[b], 8), jnp.int32) * 4
                nd_s = plsc.bitcast(lds(recs_s[b], 9), jnp.int32) * 4
                dot = rsx * rdx + rsy * rdy + rsz * rdz
                cosr = -dot * ibs * ibd
                cos = jnp.clip(cosr, -1.0, 1.0)
                dbv = bd - bs
                eh = jnp.exp(L3M * dbv * dbv * dbv)
                cc = cos - COSTHETA0
                q = D2 + cc * cc
                gang = K1 - C2G / q
                gp = 2.0 * C2G * cc / (q * q)
                inr = (cosr >= -1.0) & (cosr <= 1.0)
                A_c = jnp.where(inr, fcs * gp * eh, 0.0)
                zk = fcs * gang * eh
                P = zk * (3.0 * L3M) * dbv * dbv
                dzbs = dfcs * gang * eh - P
                alpha = -wd * A_c * ibs * ibd
                beta_d = wd * (P * ibd - A_c * cos * ibd * ibd)
                beta_s = wd * (dzbs * ibs - A_c * cos * ibs * ibs)
                cdx = alpha * rsx + beta_d * rdx
                cdy = alpha * rsy + beta_d * rdy
                cdz = alpha * rsz + beta_d * rdz
                csx = alpha * rdx + beta_s * rsx
                csy = alpha * rdy + beta_s * rsy
                csz = alpha * rdz + beta_s * rsz
                for k, (vv, ii) in enumerate((
                        (cdx, ns_d), (cdy, ns_d), (cdz, ns_d),
                        (-cdx, nd_d), (-cdy, nd_d), (-cdz, nd_d),
                        (csx, ns_s), (csy, ns_s), (csz, ns_s),
                        (-csx, nd_s), (-csy, nd_s), (-csz, nd_s))):
                    vals[b][k, sl] = vv
                    idxs[b][k, sl] = ii + (k % 3)
            for k in range(12):
                pltpu.async_copy(
                    vals[b].at[k], facc.at[idxs[b].at[k]], semsc[b], add=True)

    idx_issue(0, 0)
    idx_issue(1, 1)
    idx_wait(0, 0)
    gather_issue(0, 0)

    def loop_body(jj, carry):
        for b in (0, 1):
            j = 2 * jj + b
            o = 1 - b
            scatter_drain(j - 2, b)
            gather_wait(j, b)
            idx_issue(j + 2, b)
            idx_wait(j + 1, o)
            gather_issue(j + 1, o)
            compute_scatter(j, b)
        return carry

    lax.fori_loop(0, ITER_FWD // 2, loop_body, 0)
    scatter_drain(ITER_FWD - 2, 0)
    scatter_drain(ITER_FWD - 1, 1)

    def dir_body(i, carry):
        cnum = wid + i * NW

        @pl.when(cnum < NDIR_CHUNKS)
        def _():
            base = cnum * CHUNK
            pltpu.async_copy(table.at[pl.ds(base, CHUNK)], recs_s0, semg0).wait()
            for grp in range(CHUNK // 16):
                rows = lane + grp * 16
                sl = pl.ds(grp * 16, 16)

                def lds(c_):
                    return plsc.load_gather(
                        recs_s0, [rows, jnp.full((16,), c_, jnp.int32)])

                rx, ry, rz, w2 = lds(0), lds(1), lds(2), lds(10)
                ns = plsc.bitcast(lds(8), jnp.int32) * 4
                nd = plsc.bitcast(lds(9), jnp.int32) * 4
                cx, cy, cz = w2 * rx, w2 * ry, w2 * rz
                for k, (vv, ii) in enumerate((
                        (cx, ns), (cy, ns), (cz, ns),
                        (-cx, nd), (-cy, nd), (-cz, nd))):
                    vals0[k, sl] = vv
                    idxs0[k, sl] = ii + (k % 3)
            for k in range(6):
                pltpu.sync_copy(vals0.at[k], facc.at[idxs0.at[k]], add=True)
        return carry

    lax.fori_loop(0, ITER_DIR, dir_body, 0)
    plsc.subcore_barrier()
    pltpu.sync_copy(facc.at[pl.ds(sid * NPC * 4, NPC * 4)],
                    fout.at[pl.ds(cid * NPAD * 4 + sid * NPC * 4, NPC * 4)])


_sc_bwd = functools.partial(
    pl.kernel,
    out_type=jax.ShapeDtypeStruct((2 * NPAD * 4,), jnp.float32),
    mesh=plsc.VectorSubcoreMesh(core_axis_name="c", subcore_axis_name="s"),
    scratch_types=[
        pltpu.VMEM((CHUNK,), jnp.int32),
        pltpu.VMEM((CHUNK,), jnp.int32),
        pltpu.VMEM((CHUNK, TW), jnp.float32),
        pltpu.VMEM((CHUNK, TW), jnp.float32),
        pltpu.VMEM((12, CHUNK), jnp.float32),
        pltpu.VMEM((12, CHUNK), jnp.int32),
        pltpu.VMEM((CHUNK,), jnp.int32),
        pltpu.VMEM((CHUNK,), jnp.int32),
        pltpu.VMEM((CHUNK, TW), jnp.float32),
        pltpu.VMEM((CHUNK, TW), jnp.float32),
        pltpu.VMEM((12, CHUNK), jnp.float32),
        pltpu.VMEM((12, CHUNK), jnp.int32),
        pltpu.VMEM_SHARED((NPAD * 4,), jnp.float32),
        pltpu.SemaphoreType.DMA,
        pltpu.SemaphoreType.DMA,
        pltpu.SemaphoreType.DMA,
        pltpu.SemaphoreType.DMA,
        pltpu.SemaphoreType.DMA,
        pltpu.SemaphoreType.DMA,
    ],
    compiler_params=pltpu.CompilerParams(
        needs_layout_passes=False, use_tc_tiling_on_sc=False),
)(_sc_bwd_body)


def kernel(r, edge_index, lg_edge_index):
    lg_src = lg_edge_index[0]
    lg_dst = lg_edge_index[1]
    rxf, ryf, rzf = r[:, 0], r[:, 1], r[:, 2]
    rx2 = rxf.reshape(_ROWS, 128)
    ry2 = ryf.reshape(_ROWS, 128)
    rz2 = rzf.reshape(_ROWS, 128)
    b2, ib2, fc2, dfc2 = _tc1(rx2, ry2, rz2)
    flat = lambda a: a.reshape(-1)
    nsf = lax.bitcast_convert_type(edge_index[0], jnp.float32)
    ndf = lax.bitcast_convert_type(edge_index[1], jnp.float32)
    zcol = jnp.zeros((N_EDGES,), jnp.float32)
    bf, ibf, fcf, dfcf = flat(b2), flat(ib2), flat(fc2), flat(dfc2)
    table_f = jnp.stack(
        [rxf, ryf, rzf, bf, ibf, fcf, dfcf, zcol, nsf, ndf,
         zcol, zcol, zcol, zcol, zcol, zcol], axis=1)
    zeros_z = jnp.zeros((N_EDGES,), jnp.float32)
    zpart = _sc_fwd(table_f, lg_src, lg_dst, zeros_z).reshape(2, N_EDGES)
    z0 = zpart[0].reshape(_ROWS, 128)
    z1 = zpart[1].reshape(_ROWS, 128)
    w2d, w22d, e11 = _tc2(z0, z1, b2, ib2, fc2, dfc2)
    table_b = jnp.stack(
        [rxf, ryf, rzf, bf, ibf, fcf, dfcf, flat(w2d), nsf, ndf,
         flat(w22d), zcol, zcol, zcol, zcol, zcol], axis=1)
    zeros_f = jnp.zeros((NPAD * 4,), jnp.float32)
    fpart = _sc_bwd(table_b, lg_src, lg_dst, zeros_f)
    fsum = _tc3(fpart.reshape(2, NPAD * 4 // 128, 128))
    forces = fsum.reshape(NPAD, 4)[:N_NODES, :3]
    return (e11[0, 0], forces)
